# baseline (speedup 1.0000x reference)
import jax
import jax.numpy as jnp
from jax.experimental import pallas as pl

_EPS = 1e-5
_K = 4


def _phase1_kernel(x_ref, uw_ref, ub_ref, vw_ref, vb_ref, y_ref, stats_ref):
    f32 = jnp.float32
    xt = x_ref[0]
    J, T, C = xt.shape
    sim = jax.lax.dot_general(
        xt, xt, (((2,), (2,)), ((0,), (0,))), preferred_element_type=f32)
    ones_r = jnp.ones((J, 1, T), f32)
    ones_c = jnp.ones((J, T, 1), f32)
    bcast = lambda col: jax.lax.dot_general(
        col, ones_r, (((2,), (1,)), ((0,), (0,))), preferred_element_type=f32)
    rowsum = lambda mat: jax.lax.dot_general(
        mat, ones_c, (((2,), (1,)), ((0,), (0,))), preferred_element_type=f32)
    thr = jnp.max(sim, axis=-1, keepdims=True)
    for _ in range(_K - 1):
        ge = sim >= bcast(thr)
        cnt = rowsum(ge.astype(f32))
        nm = jnp.max(jnp.where(ge, -jnp.inf, sim), axis=-1, keepdims=True)
        thr = jnp.where(cnt < _K, nm, thr)
    adj = (sim >= bcast(thr)).astype(f32)
    deg = rowsum(adj)
    dinv = jax.lax.rsqrt(deg)
    vx = jax.lax.dot_general(
        xt, vw_ref[...], (((2,), (1,)), ((), ())),
        preferred_element_type=f32) + vb_ref[...]
    ux = jax.lax.dot_general(
        xt, uw_ref[...], (((2,), (1,)), ((), ())),
        preferred_element_type=f32) + ub_ref[...]
    agg = jax.lax.dot_general(
        adj, vx * dinv, (((2,), (1,)), ((0,), (0,))),
        preferred_element_type=f32)
    y = agg * dinv + ux
    y_ref[0] = y
    s1 = jnp.sum(jnp.sum(y, axis=-1), axis=0)
    s2 = jnp.sum(jnp.sum(y * y, axis=-1), axis=0)

    @pl.when(pl.program_id(0) == 0)
    def _init():
        stats_ref[...] = jnp.zeros_like(stats_ref)

    stats_ref[0, :] += s1
    stats_ref[1, :] += s2


def _phase2_kernel(x_ref, y_ref, sc_ref, sh_ref, o_ref):
    xt = x_ref[0]
    h = y_ref[0] * sc_ref[0] + sh_ref[0]
    o_ref[0] = jnp.maximum(xt + h, 0.0)


def kernel(x, U_w, U_b, V_w, V_b, bn_gamma, bn_beta):
    B, T, J, C = x.shape
    xt = jnp.transpose(x, (0, 2, 1, 3))
    ub = U_b.reshape(1, C)
    vb = V_b.reshape(1, C)
    y, stats = pl.pallas_call(
        _phase1_kernel,
        grid=(B,),
        in_specs=[
            pl.BlockSpec((1, J, T, C), lambda b: (b, 0, 0, 0)),
            pl.BlockSpec((C, C), lambda b: (0, 0)),
            pl.BlockSpec((1, C), lambda b: (0, 0)),
            pl.BlockSpec((C, C), lambda b: (0, 0)),
            pl.BlockSpec((1, C), lambda b: (0, 0)),
        ],
        out_specs=[
            pl.BlockSpec((1, J, T, C), lambda b: (b, 0, 0, 0)),
            pl.BlockSpec((2, T), lambda b: (0, 0)),
        ],
        out_shape=[
            jax.ShapeDtypeStruct((B, J, T, C), jnp.float32),
            jax.ShapeDtypeStruct((2, T), jnp.float32),
        ],
    )(xt, U_w, ub, V_w, vb)
    n = B * J * C
    mean = stats[0] / n
    var = stats[1] / n - mean * mean
    scale = bn_gamma * jax.lax.rsqrt(var + _EPS)
    shift = bn_beta - mean * scale
    out_t = pl.pallas_call(
        _phase2_kernel,
        grid=(B,),
        in_specs=[
            pl.BlockSpec((1, J, T, C), lambda b: (b, 0, 0, 0)),
            pl.BlockSpec((1, J, T, C), lambda b: (b, 0, 0, 0)),
            pl.BlockSpec((1, 1, T, 1), lambda b: (0, 0, 0, 0)),
            pl.BlockSpec((1, 1, T, 1), lambda b: (0, 0, 0, 0)),
        ],
        out_specs=pl.BlockSpec((1, J, T, C), lambda b: (b, 0, 0, 0)),
        out_shape=jax.ShapeDtypeStruct((B, J, T, C), jnp.float32),
    )(xt, y, scale.reshape(1, 1, T, 1), shift.reshape(1, 1, T, 1))
    return jnp.transpose(out_t, (0, 2, 1, 3))


# speedup vs baseline: 1.7593x; 1.7593x over previous
import jax
import jax.numpy as jnp
from jax.experimental import pallas as pl

_EPS = 1e-5
_K = 4


def _phase1_kernel(x_ref, uw_ref, ub_ref, vw_ref, vb_ref, y_ref, stats_ref):
    f32 = jnp.float32
    _, J, T, C = x_ref.shape
    ones_r = jnp.ones((1, T), f32)
    ones_c = jnp.ones((T, 1), f32)
    dot = lambda a, b, dims: jax.lax.dot_general(
        a, b, (dims, ((), ())), preferred_element_type=f32)
    uw = uw_ref[...]
    vw = vw_ref[...]
    ub = ub_ref[...]
    vb = vb_ref[...]
    s1 = jnp.zeros((T,), f32)
    s2 = jnp.zeros((T,), f32)
    J_ = range(J)
    xts = [x_ref[0, j] for j in J_]
    sims = [dot(xt, xt, ((1,), (1,))) for xt in xts]

    def _top4_pool(sim):
        neg = jnp.full((8, T), -jnp.inf, f32)
        m1, m2, m3, m4 = neg, neg, neg, neg
        nfull = T // 8
        ntail = T - nfull * 8
        sub_iota = jax.lax.broadcasted_iota(jnp.int32, (8, T), 0)
        for i in range(nfull + (1 if ntail else 0)):
            if i < nfull:
                v = sim[i * 8:(i + 1) * 8, :]
            else:
                v = jnp.where(sub_iota >= 8 - ntail, sim[T - 8:T, :], -jnp.inf)
            m1, r = jnp.maximum(m1, v), jnp.minimum(m1, v)
            m2, r = jnp.maximum(m2, r), jnp.minimum(m2, r)
            m3, r = jnp.maximum(m3, r), jnp.minimum(m3, r)
            m4 = jnp.maximum(m4, r)
        return jnp.concatenate([m1, m2, m3, m4], axis=0)

    def _thr_from_pool(pool):
        thr = jnp.max(pool, axis=0, keepdims=True)
        for _ in range(_K - 1):
            ge = pool >= thr
            cnt = jnp.sum(ge.astype(f32), axis=0, keepdims=True)
            nm = jnp.max(jnp.where(ge, -jnp.inf, pool), axis=0, keepdims=True)
            thr = jnp.where(cnt < _K, nm, thr)
        return thr

    thrs = [_thr_from_pool(_top4_pool(sims[j])) for j in J_]
    adjts = [(sims[j] >= thrs[j]).astype(f32) for j in J_]
    degs = [jnp.sum(a, axis=0, keepdims=True) for a in adjts]
    dinvcs = [jnp.transpose(jax.lax.rsqrt(d)) for d in degs]
    vxs = [dot(xts[j], vw, ((1,), (1,))) + vb for j in J_]
    uxs = [dot(xts[j], uw, ((1,), (1,))) + ub for j in J_]
    aggs = [dot(adjts[j], vxs[j] * dinvcs[j], ((0,), (0,))) for j in J_]
    for j in J_:
        y = aggs[j] * dinvcs[j] + uxs[j]
        y_ref[0, j] = y
        s1 += jnp.sum(y, axis=-1)
        s2 += jnp.sum(y * y, axis=-1)

    @pl.when(pl.program_id(0) == 0)
    def _init():
        stats_ref[...] = jnp.zeros_like(stats_ref)

    stats_ref[0, :] += s1
    stats_ref[1, :] += s2


def _phase2_kernel(x_ref, y_ref, sc_ref, sh_ref, o_ref):
    xt = x_ref[0]
    h = y_ref[0] * sc_ref[0] + sh_ref[0]
    o_ref[0] = jnp.maximum(xt + h, 0.0)


def kernel(x, U_w, U_b, V_w, V_b, bn_gamma, bn_beta):
    B, T, J, C = x.shape
    xt = jnp.transpose(x, (0, 2, 1, 3))
    ub = U_b.reshape(1, C)
    vb = V_b.reshape(1, C)
    y, stats = pl.pallas_call(
        _phase1_kernel,
        grid=(B,),
        in_specs=[
            pl.BlockSpec((1, J, T, C), lambda b: (b, 0, 0, 0)),
            pl.BlockSpec((C, C), lambda b: (0, 0)),
            pl.BlockSpec((1, C), lambda b: (0, 0)),
            pl.BlockSpec((C, C), lambda b: (0, 0)),
            pl.BlockSpec((1, C), lambda b: (0, 0)),
        ],
        out_specs=[
            pl.BlockSpec((1, J, T, C), lambda b: (b, 0, 0, 0)),
            pl.BlockSpec((2, T), lambda b: (0, 0)),
        ],
        out_shape=[
            jax.ShapeDtypeStruct((B, J, T, C), jnp.float32),
            jax.ShapeDtypeStruct((2, T), jnp.float32),
        ],
    )(xt, U_w, ub, V_w, vb)
    n = B * J * C
    mean = stats[0] / n
    var = stats[1] / n - mean * mean
    scale = bn_gamma * jax.lax.rsqrt(var + _EPS)
    shift = bn_beta - mean * scale
    out_t = pl.pallas_call(
        _phase2_kernel,
        grid=(B,),
        in_specs=[
            pl.BlockSpec((1, J, T, C), lambda b: (b, 0, 0, 0)),
            pl.BlockSpec((1, J, T, C), lambda b: (b, 0, 0, 0)),
            pl.BlockSpec((1, 1, T, 1), lambda b: (0, 0, 0, 0)),
            pl.BlockSpec((1, 1, T, 1), lambda b: (0, 0, 0, 0)),
        ],
        out_specs=pl.BlockSpec((1, J, T, C), lambda b: (b, 0, 0, 0)),
        out_shape=jax.ShapeDtypeStruct((B, J, T, C), jnp.float32),
    )(xt, y, scale.reshape(1, 1, T, 1), shift.reshape(1, 1, T, 1))
    return jnp.transpose(out_t, (0, 2, 1, 3))


# speedup vs baseline: 2.5203x; 1.4325x over previous
import jax
import jax.numpy as jnp
from jax.experimental import pallas as pl
from jax.experimental.pallas import tpu as pltpu

_EPS = 1e-5
_K = 4


def kernel(x, U_w, U_b, V_w, V_b, bn_gamma, bn_beta):
    B, T, J, C = x.shape
    f32 = jnp.float32

    def body(x_ref, uw_ref, ub_ref, vw_ref, vb_ref, g_ref, bt_ref,
             o_ref, y_scr, st_scr):
        step = pl.program_id(0)
        dot = lambda a, b, dims: jax.lax.dot_general(
            a, b, (dims, ((), ())), preferred_element_type=f32)

        @pl.when(step == 0)
        def _init():
            st_scr[...] = jnp.zeros_like(st_scr)

        @pl.when(step < B)
        def _phase1():
            uw = uw_ref[...]
            vw = vw_ref[...]
            ub = ub_ref[...]
            vb = vb_ref[...]
            xb = x_ref[0]

            def top4_pool(sim):
                neg = jnp.full((8, T), -jnp.inf, f32)
                m1, m2, m3, m4 = neg, neg, neg, neg
                nfull, ntail = T // 8, T % 8
                sub_iota = jax.lax.broadcasted_iota(jnp.int32, (8, T), 0)
                for i in range(nfull + (1 if ntail else 0)):
                    if i < nfull:
                        v = sim[i * 8:(i + 1) * 8, :]
                    else:
                        v = jnp.where(sub_iota >= 8 - ntail,
                                      sim[T - 8:T, :], -jnp.inf)
                    m1, r = jnp.maximum(m1, v), jnp.minimum(m1, v)
                    m2, r = jnp.maximum(m2, r), jnp.minimum(m2, r)
                    m3, r = jnp.maximum(m3, r), jnp.minimum(m3, r)
                    m4 = jnp.maximum(m4, r)
                return jnp.concatenate([m1, m2, m3, m4], axis=0)

            def thr_from_pool(pool):
                thr = jnp.max(pool, axis=0, keepdims=True)
                for _ in range(_K - 1):
                    ge = pool >= thr
                    cnt = jnp.sum(ge.astype(f32), axis=0, keepdims=True)
                    nm = jnp.max(jnp.where(ge, -jnp.inf, pool),
                                 axis=0, keepdims=True)
                    thr = jnp.where(cnt < _K, nm, thr)
                return thr

            J_ = range(J)
            xts = [xb[:, C * j:C * (j + 1)] for j in J_]
            sims = [dot(xt, xt, ((1,), (1,))) for xt in xts]
            thrs = [thr_from_pool(top4_pool(sims[j])) for j in J_]
            adjts = [(sims[j] >= thrs[j]).astype(f32) for j in J_]
            degs = [jnp.sum(a, axis=0, keepdims=True) for a in adjts]
            dinvcs = [jnp.transpose(jax.lax.rsqrt(d)) for d in degs]
            vxs = [dot(xts[j], vw, ((1,), (1,))) + vb for j in J_]
            uxs = [dot(xts[j], uw, ((1,), (1,))) + ub for j in J_]
            aggs = [dot(adjts[j], vxs[j] * dinvcs[j], ((0,), (0,)))
                    for j in J_]
            ys = [aggs[j] * dinvcs[j] + uxs[j] for j in J_]
            yb = jnp.concatenate(ys, axis=1)
            y_scr[step] = yb
            st_scr[0, :] += jnp.sum(yb, axis=-1)
            st_scr[1, :] += jnp.sum(yb * yb, axis=-1)

        @pl.when(step >= B)
        def _phase2():
            n = B * J * C
            mean = st_scr[0, :] * (1.0 / n)
            var = st_scr[1, :] * (1.0 / n) - mean * mean
            scale = g_ref[0, :] * jax.lax.rsqrt(var + _EPS)
            shift = bt_ref[0, :] - mean * scale
            scale_c = jnp.transpose(scale.reshape(1, T))
            shift_c = jnp.transpose(shift.reshape(1, T))
            h = y_scr[step - B] * scale_c + shift_c
            o_ref[0] = jnp.maximum(x_ref[0] + h, 0.0)

    xmap = lambda g: (jax.lax.rem(g, B), 0, 0)
    omap = lambda g: (jnp.maximum(g - B, 0), 0, 0)
    wmap = lambda g: (0, 0)
    out = pl.pallas_call(
        body,
        grid=(2 * B,),
        in_specs=[
            pl.BlockSpec((1, T, J * C), xmap),
            pl.BlockSpec((C, C), wmap),
            pl.BlockSpec((1, C), wmap),
            pl.BlockSpec((C, C), wmap),
            pl.BlockSpec((1, C), wmap),
            pl.BlockSpec((1, T), wmap),
            pl.BlockSpec((1, T), wmap),
        ],
        out_specs=pl.BlockSpec((1, T, J * C), omap),
        out_shape=jax.ShapeDtypeStruct((B, T, J * C), f32),
        scratch_shapes=[
            pltpu.VMEM((B, T, J * C), f32),
            pltpu.VMEM((2, T), f32),
        ],
    )(x.reshape(B, T, J * C), U_w, U_b.reshape(1, C), V_w, V_b.reshape(1, C),
      bn_gamma.reshape(1, T), bn_beta.reshape(1, T))
    return out.reshape(B, T, J, C)


# speedup vs baseline: 2.7280x; 1.0824x over previous
import jax
import jax.numpy as jnp
from jax.experimental import pallas as pl
from jax.experimental.pallas import tpu as pltpu

_EPS = 1e-5
_K = 4


def kernel(x, U_w, U_b, V_w, V_b, bn_gamma, bn_beta):
    B, T, J, C = x.shape
    f32 = jnp.float32

    def body(x_ref, uw_ref, ub_ref, vw_ref, vb_ref, g_ref, bt_ref,
             o_ref, y_scr, st_scr):
        step = pl.program_id(0)
        dot = lambda a, b, dims: jax.lax.dot_general(
            a, b, (dims, ((), ())), preferred_element_type=f32)

        @pl.when(step == 0)
        def _init():
            st_scr[...] = jnp.zeros_like(st_scr)

        PB = B // 2

        @pl.when(step < PB)
        def _phase1():
            uw = uw_ref[...]
            vw = vw_ref[...]
            ub = ub_ref[...]
            vb = vb_ref[...]

            def top4_pool(sim):
                neg = jnp.full((8, T), -jnp.inf, f32)
                m1, m2, m3, m4 = neg, neg, neg, neg
                nfull, ntail = T // 8, T % 8
                sub_iota = jax.lax.broadcasted_iota(jnp.int32, (8, T), 0)
                for i in range(nfull + (1 if ntail else 0)):
                    if i < nfull:
                        v = sim[i * 8:(i + 1) * 8, :]
                    else:
                        v = jnp.where(sub_iota >= 8 - ntail,
                                      sim[T - 8:T, :], -jnp.inf)
                    m1, r = jnp.maximum(m1, v), jnp.minimum(m1, v)
                    m2, r = jnp.maximum(m2, r), jnp.minimum(m2, r)
                    m3, r = jnp.maximum(m3, r), jnp.minimum(m3, r)
                    m4 = jnp.maximum(m4, r)
                return jnp.concatenate([m1, m2, m3, m4], axis=0)

            def thr_from_pool(pool):
                thr = jnp.max(pool, axis=0, keepdims=True)
                for _ in range(_K - 1):
                    ge = pool >= thr
                    cnt = jnp.sum(ge.astype(f32), axis=0, keepdims=True)
                    nm = jnp.max(jnp.where(ge, -jnp.inf, pool),
                                 axis=0, keepdims=True)
                    thr = jnp.where(cnt < _K, nm, thr)
                return thr

            BJ = [(bb, j) for bb in range(2) for j in range(J)]
            xts = {k: x_ref[k[0]][:, C * k[1]:C * (k[1] + 1)] for k in BJ}
            sims = {k: dot(xts[k], xts[k], ((1,), (1,))) for k in BJ}
            thrs = {k: thr_from_pool(top4_pool(sims[k])) for k in BJ}
            adjts = {k: (sims[k] >= thrs[k]).astype(f32) for k in BJ}
            degs = {k: jnp.sum(adjts[k], axis=0, keepdims=True) for k in BJ}
            dinvcs = {k: jnp.transpose(jax.lax.rsqrt(degs[k])) for k in BJ}
            vxs = {k: dot(xts[k], vw, ((1,), (1,))) + vb for k in BJ}
            uxs = {k: dot(xts[k], uw, ((1,), (1,))) + ub for k in BJ}
            aggs = {k: dot(adjts[k], vxs[k] * dinvcs[k], ((0,), (0,)))
                    for k in BJ}
            for bb in range(2):
                yb = jnp.concatenate(
                    [aggs[(bb, j)] * dinvcs[(bb, j)] + uxs[(bb, j)]
                     for j in range(J)], axis=1)
                y_scr[2 * step + bb] = yb
                st_scr[0, :] += jnp.sum(yb, axis=-1)
                st_scr[1, :] += jnp.sum(yb * yb, axis=-1)

        @pl.when(step >= PB)
        def _phase2():
            n = B * J * C
            mean = st_scr[0, :] * (1.0 / n)
            var = st_scr[1, :] * (1.0 / n) - mean * mean
            scale = g_ref[0, :] * jax.lax.rsqrt(var + _EPS)
            shift = bt_ref[0, :] - mean * scale
            scale_c = jnp.transpose(scale.reshape(1, T))
            shift_c = jnp.transpose(shift.reshape(1, T))
            for bb in range(2):
                h = y_scr[2 * (step - PB) + bb] * scale_c + shift_c
                o_ref[bb] = jnp.maximum(x_ref[bb] + h, 0.0)

    PB = B // 2
    xmap = lambda g: (jax.lax.rem(g, PB), 0, 0)
    omap = lambda g: (jnp.maximum(g - PB, 0), 0, 0)
    wmap = lambda g: (0, 0)
    out = pl.pallas_call(
        body,
        grid=(B,),
        in_specs=[
            pl.BlockSpec((2, T, J * C), xmap),
            pl.BlockSpec((C, C), wmap),
            pl.BlockSpec((1, C), wmap),
            pl.BlockSpec((C, C), wmap),
            pl.BlockSpec((1, C), wmap),
            pl.BlockSpec((1, T), wmap),
            pl.BlockSpec((1, T), wmap),
        ],
        out_specs=pl.BlockSpec((2, T, J * C), omap),
        out_shape=jax.ShapeDtypeStruct((B, T, J * C), f32),
        scratch_shapes=[
            pltpu.VMEM((B, T, J * C), f32),
            pltpu.VMEM((2, T), f32),
        ],
    )(x.reshape(B, T, J * C), U_w, U_b.reshape(1, C), V_w, V_b.reshape(1, C),
      bn_gamma.reshape(1, T), bn_beta.reshape(1, T))
    return out.reshape(B, T, J, C)


# speedup vs baseline: 2.8136x; 1.0314x over previous
import jax
import jax.numpy as jnp
from jax.experimental import pallas as pl
from jax.experimental.pallas import tpu as pltpu

_EPS = 1e-5
_K = 4


def kernel(x, U_w, U_b, V_w, V_b, bn_gamma, bn_beta):
    B, T, J, C = x.shape
    f32 = jnp.float32

    def body(x_ref, uw_ref, ub_ref, vw_ref, vb_ref, g_ref, bt_ref,
             o_ref, y_scr, st_scr):
        step = pl.program_id(0)
        dot = lambda a, b, dims: jax.lax.dot_general(
            a, b, (dims, ((), ())), preferred_element_type=f32)

        @pl.when(step == 0)
        def _init():
            st_scr[...] = jnp.zeros_like(st_scr)

        PB = B // 4

        @pl.when(step < PB)
        def _phase1():
            uw = uw_ref[...]
            vw = vw_ref[...]
            ub = ub_ref[...]
            vb = vb_ref[...]

            def top4_pool(sim):
                neg = jnp.full((8, T), -jnp.inf, f32)
                m1, m2, m3, m4 = neg, neg, neg, neg
                nfull, ntail = T // 8, T % 8
                sub_iota = jax.lax.broadcasted_iota(jnp.int32, (8, T), 0)
                for i in range(nfull + (1 if ntail else 0)):
                    if i < nfull:
                        v = sim[i * 8:(i + 1) * 8, :]
                    else:
                        v = jnp.where(sub_iota >= 8 - ntail,
                                      sim[T - 8:T, :], -jnp.inf)
                    m1, r = jnp.maximum(m1, v), jnp.minimum(m1, v)
                    m2, r = jnp.maximum(m2, r), jnp.minimum(m2, r)
                    m3, r = jnp.maximum(m3, r), jnp.minimum(m3, r)
                    m4 = jnp.maximum(m4, r)
                return jnp.concatenate([m1, m2, m3, m4], axis=0)

            def thr_from_pool(pool):
                thr = jnp.max(pool, axis=0, keepdims=True)
                for _ in range(_K - 1):
                    ge = pool >= thr
                    cnt = jnp.sum(ge.astype(f32), axis=0, keepdims=True)
                    nm = jnp.max(jnp.where(ge, -jnp.inf, pool),
                                 axis=0, keepdims=True)
                    thr = jnp.where(cnt < _K, nm, thr)
                return thr

            for g0 in range(0, 4, 2):
                BJ = [(bb, j) for bb in range(g0, g0 + 2) for j in range(J)]
                xts = {k: x_ref[k[0]][:, C * k[1]:C * (k[1] + 1)] for k in BJ}
                sims = {k: dot(xts[k], xts[k], ((1,), (1,))) for k in BJ}
                thrs = {k: thr_from_pool(top4_pool(sims[k])) for k in BJ}
                adjts = {k: (sims[k] >= thrs[k]).astype(f32) for k in BJ}
                degs = {k: jnp.sum(adjts[k], axis=0, keepdims=True)
                        for k in BJ}
                dinvcs = {k: jnp.transpose(jax.lax.rsqrt(degs[k]))
                          for k in BJ}
                vxs = {k: dot(xts[k], vw, ((1,), (1,))) + vb for k in BJ}
                uxs = {k: dot(xts[k], uw, ((1,), (1,))) + ub for k in BJ}
                aggs = {k: dot(adjts[k], vxs[k] * dinvcs[k], ((0,), (0,)))
                        for k in BJ}
                for bb in range(g0, g0 + 2):
                    yb = jnp.concatenate(
                        [aggs[(bb, j)] * dinvcs[(bb, j)] + uxs[(bb, j)]
                         for j in range(J)], axis=1)
                    y_scr[4 * step + bb] = yb
                    st_scr[0, :] += jnp.sum(yb, axis=-1)
                    st_scr[1, :] += jnp.sum(yb * yb, axis=-1)

        @pl.when(step >= PB)
        def _phase2():
            n = B * J * C
            mean = st_scr[0, :] * (1.0 / n)
            var = st_scr[1, :] * (1.0 / n) - mean * mean
            scale = g_ref[0, :] * jax.lax.rsqrt(var + _EPS)
            shift = bt_ref[0, :] - mean * scale
            scale_c = jnp.transpose(scale.reshape(1, T))
            shift_c = jnp.transpose(shift.reshape(1, T))
            for bb in range(4):
                h = y_scr[4 * (step - PB) + bb] * scale_c + shift_c
                o_ref[bb] = jnp.maximum(x_ref[bb] + h, 0.0)

    PB = B // 4
    xmap = lambda g: (jax.lax.rem(g, PB), 0, 0)
    omap = lambda g: (jnp.maximum(g - PB, 0), 0, 0)
    wmap = lambda g: (0, 0)
    out = pl.pallas_call(
        body,
        grid=(B // 2,),
        in_specs=[
            pl.BlockSpec((4, T, J * C), xmap),
            pl.BlockSpec((C, C), wmap),
            pl.BlockSpec((1, C), wmap),
            pl.BlockSpec((C, C), wmap),
            pl.BlockSpec((1, C), wmap),
            pl.BlockSpec((1, T), wmap),
            pl.BlockSpec((1, T), wmap),
        ],
        out_specs=pl.BlockSpec((4, T, J * C), omap),
        out_shape=jax.ShapeDtypeStruct((B, T, J * C), f32),
        scratch_shapes=[
            pltpu.VMEM((B, T, J * C), f32),
            pltpu.VMEM((2, T), f32),
        ],
    )(x.reshape(B, T, J * C), U_w, U_b.reshape(1, C), V_w, V_b.reshape(1, C),
      bn_gamma.reshape(1, T), bn_beta.reshape(1, T))
    return out.reshape(B, T, J, C)
